# baseline (speedup 1.0000x reference)
import functools

import jax
import jax.numpy as jnp
from jax import lax
from jax.experimental import pallas as pl
from jax.experimental.pallas import tpu as pltpu

_X = 64
_Y = 64
_N = _X * _Y
_DIM = 128
_NUM_ITERS = 100.0
_LEARNING_RADIUS = 0.5
_SIGMA = 32.0
_COGNITIVE = 0.01
_SOCIAL = 0.1
_INERTIA = 0.001

_BLK = 128
_NBLK = _N // _BLK


def _som_body(params_ref, in_ref, pfull_ref, pblk_ref, vblk_ref, r1_ref, r2_ref,
              outp_ref, outv_ref, smem, best_row):
    s = pl.program_id(0)

    @pl.when(s == 0)
    def _argmin_phase():
        diff = in_ref[0:1, :] - pfull_ref[:, :] + 1e-6
        ss = jnp.sum(diff * diff, axis=1, keepdims=True)
        m = jnp.min(ss)
        rows = lax.broadcasted_iota(jnp.int32, (_N, 1), 0)
        idx = jnp.min(jnp.where(ss == m, rows, _N))
        smem[0] = idx
        smem[1] = idx // _Y
        smem[2] = idx % _Y
        best_row[0:1, :] = pfull_ref[pl.ds(idx, 1), :]

    @pl.when(s > 0)
    def _block_phase():
        b = s - 1
        lr = params_ref[0]
        denom = params_ref[1]
        bx = smem[1]
        by = smem[2]
        j = lax.broadcasted_iota(jnp.int32, (1, _N), 1)
        jx = j // _Y
        jy = j % _Y
        d2j = ((jx - bx) * (jx - bx) + (jy - by) * (jy - by)).astype(jnp.float32)
        vj = jnp.exp(-(d2j / denom))
        i = b * _BLK + lax.broadcasted_iota(jnp.int32, (_BLK, 1), 0)
        ix = i // _Y
        iy = i % _Y
        d2i = ((ix - bx) * (ix - bx) + (iy - by) * (iy - by)).astype(jnp.float32)
        vi = jnp.exp(-(d2i / denom))

        maskf = jnp.where(vj <= vi + lr, 1.0, 0.0).astype(jnp.float32)
        counts = jnp.sum(maskf, axis=1, keepdims=True)
        numer = lax.dot_general(
            maskf, pfull_ref[:, :], (((1,), (0,)), ((), ())),
            preferred_element_type=jnp.float32)
        centroid = numer / counts

        p = pblk_ref[:, :]
        v_cog = _COGNITIVE * r1_ref[:, :] * (centroid - p)
        v_soc = _SOCIAL * r2_ref[:, :] * (best_row[0:1, :] - p)
        v_upd = _INERTIA * vblk_ref[:, :] + v_cog + v_soc
        upd = (1.0 - vi) <= lr
        outv_ref[:, :] = jnp.where(upd, v_upd, vblk_ref[:, :])
        outp_ref[:, :] = jnp.where(upd, p + v_upd, p)


@functools.partial(jax.jit)
def _som_tc(params, input_vec, particles, velocities, r1, r2):
    blk_idx = lambda s: (jnp.maximum(s - 1, 0), 0)
    return pl.pallas_call(
        _som_body,
        grid=(_NBLK + 1,),
        in_specs=[
            pl.BlockSpec(memory_space=pltpu.SMEM),
            pl.BlockSpec((1, _DIM), lambda s: (0, 0)),
            pl.BlockSpec((_N, _DIM), lambda s: (0, 0)),
            pl.BlockSpec((_BLK, _DIM), blk_idx),
            pl.BlockSpec((_BLK, _DIM), blk_idx),
            pl.BlockSpec((_BLK, _DIM), blk_idx),
            pl.BlockSpec((_BLK, _DIM), blk_idx),
        ],
        out_specs=[
            pl.BlockSpec((_BLK, _DIM), blk_idx),
            pl.BlockSpec((_BLK, _DIM), blk_idx),
        ],
        out_shape=[
            jax.ShapeDtypeStruct((_N, _DIM), jnp.float32),
            jax.ShapeDtypeStruct((_N, _DIM), jnp.float32),
        ],
        scratch_shapes=[
            pltpu.SMEM((4,), jnp.int32),
            pltpu.VMEM((1, _DIM), jnp.float32),
        ],
    )(params, input_vec.reshape(1, _DIM), particles, particles, velocities,
      r1, r2)


def kernel(input_vec, iter_num, particles, velocities, grid_locations, r1, r2):
    del grid_locations
    decay = 1.0 - iter_num / _NUM_ITERS
    lr_decay = jnp.float32(_LEARNING_RADIUS * decay)
    sigma_decay = jnp.float32(_SIGMA * decay)
    params = jnp.stack([lr_decay, sigma_decay * sigma_decay])
    new_p, new_v = _som_tc(params, input_vec, particles, velocities, r1, r2)
    return new_p, new_v
